# baseline (speedup 1.0000x reference)
import functools

import jax
import jax.numpy as jnp
from jax import lax
from jax.experimental import pallas as pl
from jax.experimental.pallas import tpu as pltpu
from jax.experimental.pallas import tpu_sc as plsc

R, D, N = 262144, 64, 16384
DP = 128
NC, NS, L = 2, 16, 16
W = NC * NS
CPT = R // W
CW = 256
NSLAB = CPT // CW
IBLK = 4096
CAP = N + 64

_mesh = plsc.VectorSubcoreMesh(
    core_axis_name="c", subcore_axis_name="s", num_cores=NC, num_subcores=NS
)


@functools.partial(
    pl.kernel,
    out_type=jax.ShapeDtypeStruct((D, R), jnp.float32),
    mesh=_mesh,
    scratch_types=[
        pltpu.VMEM((IBLK,), jnp.int32),
        pltpu.VMEM((CAP,), jnp.int32),
        pltpu.VMEM((CAP,), jnp.int32),
        pltpu.VMEM((2, D, CW), jnp.float32),
        pltpu.VMEM((64, DP), jnp.float32),
        pltpu.VMEM((64,), jnp.int32),
        pltpu.VMEM((64 + L,), jnp.int32),
        pltpu.SemaphoreType.DMA,
        pltpu.SemaphoreType.DMA,
        pltpu.SemaphoreType.DMA,
        pltpu.SemaphoreType.DMA,
        pltpu.SemaphoreType.DMA,
    ],
    compiler_params=pltpu.CompilerParams(needs_layout_passes=False),
)
def _scatter_add(idx_hbm, at_hbm, bp_hbm, out_hbm,
                 idx_v, buf1, buf2, slabs, stage, bpos64, lcol64,
                 gsem, isem0, isem1, osem0, osem1):
    c = lax.axis_index("c")
    s = lax.axis_index("s")
    w = s * NC + c
    tile_base = w * CPT
    iota = lax.iota(jnp.int32, L)
    isem = (isem0, isem1)
    osem = (osem0, osem1)

    def blk(b, cnt):
        pltpu.sync_copy(idx_hbm.at[pl.ds(b * IBLK, IBLK)], idx_v)

        def f1(v, cnt):
            for u in range(2):
                iv = idx_v[pl.ds((2 * v + u) * L, L)]
                li = iv - tile_base
                m = (li >= 0) & (li < CPT)
                packed = li * N + (b * IBLK + (2 * v + u) * L + iota)
                plsc.store_compressed(buf1.at[pl.ds(cnt, L)], packed, mask=m)
                cnt = cnt + plsc.all_reduce_population_count(m)[0]
            return cnt

        return lax.fori_loop(0, IBLK // L // 2, f1, cnt)

    cnt1 = lax.fori_loop(0, N // IBLK, blk, jnp.int32(0))
    nv1 = (cnt1 + L - 1) // L

    def compute_slab(j, p):
        col0 = j * CW
        lo = col0 * N
        hi = (col0 + CW) * N
        sub = slabs.at[p]

        def f2(v, cnt):
            pv = buf1[pl.ds(v * L, L)]
            m = (pv >= lo) & (pv < hi) & (v * L + iota < cnt1)
            plsc.store_compressed(buf2.at[pl.ds(cnt, L)], pv, mask=m)
            return cnt + plsc.all_reduce_population_count(m)[0]

        cnt2 = lax.fori_loop(0, nv1, f2, jnp.int32(0))

        def g(k, carry2):
            base = k * 64
            for u in range(4):
                pv = buf2[pl.ds(base + u * L, L)]
                bpos64[pl.ds(u * L, L)] = pv & (N - 1)
                lcol64[pl.ds(u * L, L)] = (pv >> 14) - col0
            pltpu.async_copy(bp_hbm.at[bpos64], stage, gsem).wait()
            valid = jnp.minimum(cnt2 - base, 64)

            def ubody(t, carry3):
                rc = lcol64[pl.ds(t, L)][0]
                cols = jnp.full((L,), 0, jnp.int32) + rc
                for q in range(D // L):
                    plsc.addupdate_scatter(
                        sub, [q * L + iota, cols],
                        stage[t, pl.ds(q * L, L)],
                    )
                return carry3

            lax.fori_loop(0, valid, ubody, jnp.int32(0))
            return carry2

        ng = (cnt2 + 63) // 64
        lax.fori_loop(0, ng, g, jnp.int32(0))

    def issue_in(j, p):
        return pltpu.async_copy(
            at_hbm.at[:, pl.ds(tile_base + j * CW, CW)], slabs.at[p], isem[p]
        )

    def issue_out(j, p):
        return pltpu.async_copy(
            slabs.at[p], out_hbm.at[:, pl.ds(tile_base + j * CW, CW)], osem[p]
        )

    d_in = [None, None]
    d_out = [None, None]
    d_in[0] = issue_in(0, 0)
    for j in range(NSLAB):
        p = j & 1
        if j + 1 < NSLAB:
            if d_out[1 - p] is not None:
                d_out[1 - p].wait()
            d_in[1 - p] = issue_in(j + 1, 1 - p)
        d_in[p].wait()
        compute_slab(j, p)
        d_out[p] = issue_out(j, p)
    d_out[0].wait()
    d_out[1].wait()


def kernel(index, A, B):
    b_pad = jnp.pad(B, ((0, 0), (0, DP - D)))
    out_t = _scatter_add(index.astype(jnp.int32), A.T, b_pad)
    return out_t.T


# speedup vs baseline: 1.2472x; 1.2472x over previous
import functools

import jax
import jax.numpy as jnp
from jax import lax
from jax.experimental import pallas as pl
from jax.experimental.pallas import tpu as pltpu
from jax.experimental.pallas import tpu_sc as plsc

R, D, N = 262144, 64, 16384
DP = 128
NC, NS, L = 2, 16, 16
W = NC * NS
CPT = R // W
CW = 512
NSLAB = CPT // CW
IBLK = 4096
CAP = N + 64

_mesh = plsc.VectorSubcoreMesh(
    core_axis_name="c", subcore_axis_name="s", num_cores=NC, num_subcores=NS
)


@functools.partial(
    pl.kernel,
    out_type=jax.ShapeDtypeStruct((D, R), jnp.float32),
    mesh=_mesh,
    scratch_types=[
        pltpu.VMEM((IBLK,), jnp.int32),
        pltpu.VMEM((CAP,), jnp.int32),
        pltpu.VMEM((CAP,), jnp.int32),
        pltpu.VMEM((2, D, CW), jnp.float32),
        pltpu.VMEM((L, DP), jnp.float32),
        pltpu.VMEM((2 * L,), jnp.int32),
        pltpu.SemaphoreType.DMA,
        pltpu.SemaphoreType.DMA,
        pltpu.SemaphoreType.DMA,
        pltpu.SemaphoreType.DMA,
        pltpu.SemaphoreType.DMA,
    ],
    compiler_params=pltpu.CompilerParams(needs_layout_passes=False),
)
def _scatter_add(idx_hbm, at_hbm, bp_hbm, out_hbm,
                 idx_v, buf1, buf2, slabs, stage, rowbuf,
                 gsem, isem0, isem1, osem0, osem1):
    c = lax.axis_index("c")
    s = lax.axis_index("s")
    w = s * NC + c
    tile_base = w * CPT
    iota = lax.iota(jnp.int32, L)
    isem = (isem0, isem1)
    osem = (osem0, osem1)

    def blk(b, cnt):
        pltpu.sync_copy(idx_hbm.at[pl.ds(b * IBLK, IBLK)], idx_v)

        def f1(v, cnt):
            for u in range(2):
                iv = idx_v[pl.ds((2 * v + u) * L, L)]
                li = iv - tile_base
                m = (li >= 0) & (li < CPT)
                packed = li * N + (b * IBLK + (2 * v + u) * L + iota)
                plsc.store_compressed(buf1.at[pl.ds(cnt, L)], packed, mask=m)
                cnt = cnt + plsc.all_reduce_population_count(m)[0]
            return cnt

        return lax.fori_loop(0, IBLK // L // 2, f1, cnt)

    cnt1 = lax.fori_loop(0, N // IBLK, blk, jnp.int32(0))
    nv1 = (cnt1 + L - 1) // L

    def compute_slab(j, p):
        col0 = j * CW
        lo = col0 * N
        hi = (col0 + CW) * N
        sub = slabs.at[p]

        def f2(v, cnt):
            for u in range(2):
                pv = buf1[pl.ds((2 * v + u) * L, L)]
                m = (pv >= lo) & (pv < hi) & ((2 * v + u) * L + iota < cnt1)
                plsc.store_compressed(buf2.at[pl.ds(cnt, L)], pv, mask=m)
                cnt = cnt + plsc.all_reduce_population_count(m)[0]
            return cnt

        cnt2 = lax.fori_loop(0, (nv1 + 1) // 2, f2, jnp.int32(0))
        buf2[pl.ds(cnt2, L)] = jnp.full((L,), -1, jnp.int32)

        def g(k, carry2):
            pv = buf2[pl.ds(k * L, L)]
            bpos = jnp.where(pv < 0, -1, pv & (N - 1))
            gidx = plsc.Indices(bpos, ignored_value=-1)
            pltpu.async_copy(bp_hbm.at[gidx], stage, gsem).wait()
            lcol = (pv >> 14) - col0
            rowbuf[pl.ds(0, L)] = lcol
            valid = jnp.minimum(cnt2 - k * L, L)

            def ubody(t, carry3):
                rc = rowbuf[pl.ds(t, L)][0]
                cols = jnp.full((L,), 0, jnp.int32) + rc
                for q in range(D // L):
                    plsc.addupdate_scatter(
                        sub, [q * L + iota, cols],
                        stage[t, pl.ds(q * L, L)],
                    )
                return carry3

            lax.fori_loop(0, valid, ubody, jnp.int32(0))
            return carry2

        ng = (cnt2 + L - 1) // L
        lax.fori_loop(0, ng, g, jnp.int32(0))

    def issue_in(j, p):
        return pltpu.async_copy(
            at_hbm.at[:, pl.ds(tile_base + j * CW, CW)], slabs.at[p], isem[p]
        )

    def issue_out(j, p):
        return pltpu.async_copy(
            slabs.at[p], out_hbm.at[:, pl.ds(tile_base + j * CW, CW)], osem[p]
        )

    d_in = [None, None]
    d_out = [None, None]
    d_in[0] = issue_in(0, 0)
    for j in range(NSLAB):
        p = j & 1
        if j + 1 < NSLAB:
            if d_out[1 - p] is not None:
                d_out[1 - p].wait()
            d_in[1 - p] = issue_in(j + 1, 1 - p)
        d_in[p].wait()
        compute_slab(j, p)
        d_out[p] = issue_out(j, p)
    d_out[0].wait()
    d_out[1].wait()


def kernel(index, A, B):
    b_pad = jnp.pad(B, ((0, 0), (0, DP - D)))
    out_t = _scatter_add(index.astype(jnp.int32), A.T, b_pad)
    return out_t.T


# speedup vs baseline: 1.3141x; 1.0536x over previous
import functools

import jax
import jax.numpy as jnp
from jax import lax
from jax.experimental import pallas as pl
from jax.experimental.pallas import tpu as pltpu
from jax.experimental.pallas import tpu_sc as plsc

R, D, N = 262144, 64, 16384
DP = 128
NC, NS, L = 2, 16, 16
W = NC * NS
CPT = R // W
CW = 256
NSLAB = CPT // CW
IBLK = 4096
CAP = N + 64

_mesh = plsc.VectorSubcoreMesh(
    core_axis_name="c", subcore_axis_name="s", num_cores=NC, num_subcores=NS
)


@functools.partial(
    pl.kernel,
    out_type=jax.ShapeDtypeStruct((D, R), jnp.float32),
    mesh=_mesh,
    scratch_types=[
        pltpu.VMEM((IBLK,), jnp.int32),
        pltpu.VMEM((CAP,), jnp.int32),
        pltpu.VMEM((CAP,), jnp.int32),
        pltpu.VMEM((3, D, CW), jnp.float32),
        pltpu.VMEM((L, DP), jnp.float32),
        pltpu.VMEM((2 * L,), jnp.int32),
        pltpu.SemaphoreType.DMA,
        pltpu.SemaphoreType.DMA,
        pltpu.SemaphoreType.DMA,
        pltpu.SemaphoreType.DMA,
        pltpu.SemaphoreType.DMA,
        pltpu.SemaphoreType.DMA,
        pltpu.SemaphoreType.DMA,
    ],
    compiler_params=pltpu.CompilerParams(needs_layout_passes=False),
)
def _scatter_add(idx_hbm, at_hbm, bp_hbm, out_hbm,
                 idx_v, buf1, buf2, slabs, stage, rowbuf,
                 gsem, isem0, isem1, isem2, osem0, osem1, osem2):
    c = lax.axis_index("c")
    s = lax.axis_index("s")
    w = s * NC + c
    tile_base = w * CPT
    iota = lax.iota(jnp.int32, L)
    isem = (isem0, isem1, isem2)
    osem = (osem0, osem1, osem2)

    def issue_in(j, p):
        return pltpu.async_copy(
            at_hbm.at[:, pl.ds(tile_base + j * CW, CW)], slabs.at[p], isem[p]
        )

    def issue_out(j, p):
        return pltpu.async_copy(
            slabs.at[p], out_hbm.at[:, pl.ds(tile_base + j * CW, CW)], osem[p]
        )

    d_in = [issue_in(0, 0), issue_in(1, 1), None]
    d_out = [None, None, None]

    def blk(b, cnt):
        pltpu.sync_copy(idx_hbm.at[pl.ds(b * IBLK, IBLK)], idx_v)

        def f1(v, cnt):
            for u in range(2):
                iv = idx_v[pl.ds((2 * v + u) * L, L)]
                li = iv - tile_base
                m = (li >= 0) & (li < CPT)
                packed = li * N + (b * IBLK + (2 * v + u) * L + iota)
                plsc.store_compressed(buf1.at[pl.ds(cnt, L)], packed, mask=m)
                cnt = cnt + plsc.all_reduce_population_count(m)[0]
            return cnt

        return lax.fori_loop(0, IBLK // L // 2, f1, cnt)

    cnt1 = lax.fori_loop(0, N // IBLK, blk, jnp.int32(0))
    nv1 = (cnt1 + L - 1) // L

    def compute_slab(j, p):
        col0 = j * CW
        lo = col0 * N
        hi = (col0 + CW) * N
        sub = slabs.at[p]

        def f2(v, cnt):
            for u in range(2):
                pv = buf1[pl.ds((2 * v + u) * L, L)]
                m = (pv >= lo) & (pv < hi) & ((2 * v + u) * L + iota < cnt1)
                plsc.store_compressed(buf2.at[pl.ds(cnt, L)], pv, mask=m)
                cnt = cnt + plsc.all_reduce_population_count(m)[0]
            return cnt

        cnt2 = lax.fori_loop(0, (nv1 + 1) // 2, f2, jnp.int32(0))
        buf2[pl.ds(cnt2, L)] = jnp.full((L,), -1, jnp.int32)

        def g(k, carry2):
            pv = buf2[pl.ds(k * L, L)]
            bpos = jnp.where(pv < 0, -1, pv & (N - 1))
            gidx = plsc.Indices(bpos, ignored_value=-1)
            pltpu.async_copy(bp_hbm.at[gidx], stage, gsem).wait()
            lcol = (pv >> 14) - col0
            rowbuf[pl.ds(0, L)] = lcol
            valid = jnp.minimum(cnt2 - k * L, L)

            def ubody(t, carry3):
                rc = rowbuf[pl.ds(t, L)][0]
                cols = jnp.full((L,), 0, jnp.int32) + rc
                for q in range(D // L):
                    plsc.addupdate_scatter(
                        sub, [q * L + iota, cols],
                        stage[t, pl.ds(q * L, L)],
                    )
                return carry3

            lax.fori_loop(0, valid, ubody, jnp.int32(0))
            return carry2

        ng = (cnt2 + L - 1) // L
        lax.fori_loop(0, ng, g, jnp.int32(0))

    for j in range(NSLAB):
        p = j % 3
        d_in[p].wait()
        compute_slab(j, p)
        d_out[p] = issue_out(j, p)
        if j + 2 < NSLAB:
            pn = (j + 2) % 3
            if d_out[pn] is not None:
                d_out[pn].wait()
            d_in[pn] = issue_in(j + 2, pn)
    for p in range(3):
        if d_out[p] is not None:
            d_out[p].wait()


def kernel(index, A, B):
    b_pad = jnp.pad(B, ((0, 0), (0, DP - D)))
    out_t = _scatter_add(index.astype(jnp.int32), A.T, b_pad)
    return out_t.T


# speedup vs baseline: 1.5530x; 1.1818x over previous
import functools

import jax
import jax.numpy as jnp
from jax import lax
from jax.experimental import pallas as pl
from jax.experimental.pallas import tpu as pltpu
from jax.experimental.pallas import tpu_sc as plsc

R, D, N = 262144, 64, 16384
DP = 128
NC, NS, L = 2, 16, 16
W = NC * NS
CPT = R // W
CW = 256
NSLAB = CPT // CW
IBLK = 4096
CAP = N + 64

_mesh = plsc.VectorSubcoreMesh(
    core_axis_name="c", subcore_axis_name="s", num_cores=NC, num_subcores=NS
)


@functools.partial(
    pl.kernel,
    out_type=jax.ShapeDtypeStruct((D, R), jnp.float32),
    mesh=_mesh,
    scratch_types=[
        pltpu.VMEM((IBLK,), jnp.int32),
        pltpu.VMEM((CAP,), jnp.int32),
        pltpu.VMEM((2 * CAP,), jnp.int32),
        pltpu.VMEM((3, D, CW), jnp.float32),
        pltpu.VMEM((2 * L, DP), jnp.float32),
        pltpu.VMEM((2 * L,), jnp.int32),
        pltpu.SemaphoreType.DMA,
        pltpu.SemaphoreType.DMA,
        pltpu.SemaphoreType.DMA,
        pltpu.SemaphoreType.DMA,
        pltpu.SemaphoreType.DMA,
        pltpu.SemaphoreType.DMA,
        pltpu.SemaphoreType.DMA,
        pltpu.SemaphoreType.DMA,
    ],
    compiler_params=pltpu.CompilerParams(needs_layout_passes=False),
)
def _scatter_add(idx_hbm, at_hbm, bp_hbm, out_hbm,
                 idx_v, buf1, buf2, slabs, stage, rowbuf,
                 gsem0, gsem1, isem0, isem1, isem2, osem0, osem1, osem2):
    c = lax.axis_index("c")
    s = lax.axis_index("s")
    w = s * NC + c
    tile_base = w * CPT
    iota = lax.iota(jnp.int32, L)
    isem = (isem0, isem1, isem2)
    osem = (osem0, osem1, osem2)

    def issue_in(j, p):
        return pltpu.async_copy(
            at_hbm.at[:, pl.ds(tile_base + j * CW, CW)], slabs.at[p], isem[p]
        )

    def issue_out(j, p):
        return pltpu.async_copy(
            slabs.at[p], out_hbm.at[:, pl.ds(tile_base + j * CW, CW)], osem[p]
        )

    d_in = [issue_in(0, 0), issue_in(1, 1), None]
    d_out = [None, None, None]

    def blk(b, cnt):
        pltpu.sync_copy(idx_hbm.at[pl.ds(b * IBLK, IBLK)], idx_v)

        def f1(v, cnt):
            for u in range(2):
                iv = idx_v[pl.ds((2 * v + u) * L, L)]
                li = iv - tile_base
                m = (li >= 0) & (li < CPT)
                packed = li * N + (b * IBLK + (2 * v + u) * L + iota)
                plsc.store_compressed(buf1.at[pl.ds(cnt, L)], packed, mask=m)
                cnt = cnt + plsc.all_reduce_population_count(m)[0]
            return cnt

        return lax.fori_loop(0, IBLK // L // 2, f1, cnt)

    cnt1 = lax.fori_loop(0, N // IBLK, blk, jnp.int32(0))
    nv1 = (cnt1 + L - 1) // L

    gsem = (gsem0, gsem1)

    def prep_slab(j, q):
        col0 = j * CW
        lo = col0 * N
        hi = (col0 + CW) * N
        qb = q * CAP

        def f2(v, cnt):
            for u in range(2):
                pv = buf1[pl.ds((2 * v + u) * L, L)]
                m = (pv >= lo) & (pv < hi) & ((2 * v + u) * L + iota < cnt1)
                plsc.store_compressed(buf2.at[pl.ds(qb + cnt, L)], pv, mask=m)
                cnt = cnt + plsc.all_reduce_population_count(m)[0]
            return cnt

        cnt2 = lax.fori_loop(0, (nv1 + 1) // 2, f2, jnp.int32(0))
        buf2[pl.ds(qb + cnt2, L)] = jnp.full((L,), -1, jnp.int32)
        pv = buf2[pl.ds(qb, L)]
        bpos = jnp.where(pv < 0, jnp.where(iota > 0, -1, iota), pv & (N - 1))
        gidx = plsc.Indices(bpos, ignored_value=-1)
        d = pltpu.async_copy(bp_hbm.at[gidx],
                             stage.at[pl.ds(q * L, L)], gsem[q])
        return cnt2, d

    def process_slab(j, p, q, cnt2, d0):
        col0 = j * CW
        sub = slabs.at[p]

        def g(k, carry2):
            pv = buf2[pl.ds(q * CAP + k * L, L)]
            is_first = k == 0

            @pl.when(jnp.logical_not(is_first))
            def _():
                bpos = jnp.where(pv < 0, -1, pv & (N - 1))
                gidx = plsc.Indices(bpos, ignored_value=-1)
                pltpu.async_copy(bp_hbm.at[gidx],
                                 stage.at[pl.ds(q * L, L)], gsem[q]).wait()

            lcol = (pv >> 14) - col0
            rowbuf[pl.ds(0, L)] = lcol
            valid = jnp.minimum(cnt2 - k * L, L)

            def ubody(t, carry3):
                rc = rowbuf[pl.ds(t, L)][0]
                cols = jnp.full((L,), 0, jnp.int32) + rc
                for qq in range(D // L):
                    plsc.addupdate_scatter(
                        sub, [qq * L + iota, cols],
                        stage[q * L + t, pl.ds(qq * L, L)],
                    )
                return carry3

            lax.fori_loop(0, valid, ubody, jnp.int32(0))
            return carry2

        ng = (cnt2 + L - 1) // L
        d0.wait()
        lax.fori_loop(0, ng, g, jnp.int32(0))

    slab_meta = prep_slab(0, 0)
    for j in range(NSLAB):
        p = j % 3
        d_in[p].wait()
        cnt2_j, d0_j = slab_meta
        if j + 1 < NSLAB:
            slab_meta = prep_slab(j + 1, (j + 1) & 1)
        process_slab(j, p, j & 1, cnt2_j, d0_j)
        d_out[p] = issue_out(j, p)
        if j + 2 < NSLAB:
            pn = (j + 2) % 3
            if d_out[pn] is not None:
                d_out[pn].wait()
            d_in[pn] = issue_in(j + 2, pn)
    for p in range(3):
        if d_out[p] is not None:
            d_out[p].wait()


def kernel(index, A, B):
    b_pad = jnp.pad(B, ((0, 0), (0, DP - D)))
    out_t = _scatter_add(index.astype(jnp.int32), A.T, b_pad)
    return out_t.T
